# baseline (speedup 1.0000x reference)
import jax
import jax.numpy as jnp
from jax import lax
from jax.experimental import pallas as pl
from jax.experimental.pallas import tpu as pltpu
from jax.experimental.pallas import tpu_sc as plsc

N_FIELDS = 26
VOCAB = 100000
EMB = 32
BATCH = 16384

NC = 2
NS = 16
NW = NC * NS
B_PER_W = BATCH // NW
ROWS_PER_CHUNK = 4
CHUNK = ROWS_PER_CHUNK * N_FIELDS
NCHUNKS = B_PER_W // ROWS_PER_CHUNK
NBUF = 2
NGRP = NCHUNKS // NBUF
LANES = 16


def _tec_body(x_hbm, tbl_hbm, out_hbm, x_v, idx_v, col_v, gbuf, acc,
              ld_sem, g_sems):
  wid = lax.axis_index("s") * NC + lax.axis_index("c")

  pltpu.async_copy(x_hbm.at[pl.ds(wid * NCHUNKS, NCHUNKS)], x_v,
                   ld_sem).wait()

  off0 = lax.iota(jnp.int32, LANES) * VOCAB
  off1 = off0 + 10 * VOCAB
  three = jnp.full((LANES,), 3, jnp.int32)

  def _build_and_fire(ch, b):
    for r in range(ROWS_PER_CHUNK):
      v0 = x_v[ch, pl.ds(32 * r, LANES)] + off0
      v1 = x_v[ch, pl.ds(32 * r + 10, LANES)] + off1
      idx_v[b, pl.ds(r * N_FIELDS, LANES)] = lax.shift_right_logical(v0, 2)
      idx_v[b, pl.ds(r * N_FIELDS + 10, LANES)] = lax.shift_right_logical(v1, 2)
      col_v[b, pl.ds(r * N_FIELDS, LANES)] = lax.shift_left(v0 & three, 5)
      col_v[b, pl.ds(r * N_FIELDS + 10, LANES)] = lax.shift_left(v1 & three, 5)
    pltpu.async_copy(tbl_hbm.at[idx_v.at[b]],
                     gbuf.at[pl.ds(b * CHUNK, CHUNK)], g_sems.at[b])

  for b in range(NBUF):
    _build_and_fire(b, b)

  def _grp(g, c):
    for b in range(NBUF):
      ch = g * NBUF + b
      pltpu.make_async_copy(tbl_hbm.at[idx_v.at[b]],
                            gbuf.at[pl.ds(b * CHUNK, CHUNK)],
                            g_sems.at[b]).wait()

      for r in range(ROWS_PER_CHUNK):
        src = b * CHUNK + r * N_FIELDS
        pos = r * N_FIELDS
        cv0 = col_v[b, pl.ds(pos, LANES)]
        cv1 = col_v[b, pl.ds(pos + 10, LANES)]
        cols = [cv0[i] if i < LANES else cv1[i - 10] for i in range(N_FIELDS)]
        p0 = [gbuf[src + i, pl.ds(cols[i], LANES)] for i in range(4)]
        p1 = [gbuf[src + i, pl.ds(cols[i] + LANES, LANES)] for i in range(4)]
        for i in range(4, N_FIELDS):
          p0[i % 4] += gbuf[src + i, pl.ds(cols[i], LANES)]
          p1[i % 4] += gbuf[src + i, pl.ds(cols[i] + LANES, LANES)]
        acc[ch, pl.ds(32 * r, LANES)] = (p0[0] + p0[1]) + (p0[2] + p0[3])
        acc[ch, pl.ds(32 * r + LANES, LANES)] = (
            (p1[0] + p1[1]) + (p1[2] + p1[3]))

      @pl.when(g < NGRP - 1)
      def _():
        _build_and_fire(ch + NBUF, b)

    return c

  lax.fori_loop(0, NGRP, _grp, 0, unroll=False)

  pltpu.async_copy(acc, out_hbm.at[pl.ds(wid * NCHUNKS, NCHUNKS)],
                   ld_sem).wait()


@jax.jit
def kernel(x, tables):
  t3 = jax.lax.optimization_barrier(tables.reshape(N_FIELDS, VOCAB // 4, 128))
  tbl128 = t3.reshape(N_FIELDS * VOCAB * EMB // 128, 128)
  xp = jnp.pad(x, ((0, 0), (0, 32 - N_FIELDS))).reshape(BATCH // 4, 128)

  mesh = plsc.VectorSubcoreMesh(core_axis_name="c", subcore_axis_name="s")
  f = pl.kernel(
      _tec_body,
      out_type=jax.ShapeDtypeStruct((BATCH // 4, 128), jnp.float32),
      mesh=mesh,
      compiler_params=pltpu.CompilerParams(use_tc_tiling_on_sc=True),
      scratch_types=[
          pltpu.VMEM((NCHUNKS, 128), jnp.int32),
          pltpu.VMEM((NBUF, CHUNK), jnp.int32),
          pltpu.VMEM((NBUF, CHUNK), jnp.int32),
          pltpu.VMEM((NBUF * CHUNK, 128), jnp.float32),
          pltpu.VMEM((NCHUNKS, 128), jnp.float32),
          pltpu.SemaphoreType.DMA,
          pltpu.SemaphoreType.DMA((NBUF,)),
      ],
  )
  return f(xp, tbl128).reshape(BATCH, EMB)


# speedup vs baseline: 1.0400x; 1.0400x over previous
import jax
import jax.numpy as jnp
from jax import lax
from jax.experimental import pallas as pl
from jax.experimental.pallas import tpu as pltpu
from jax.experimental.pallas import tpu_sc as plsc

N_FIELDS = 26
VOCAB = 100000
EMB = 32
BATCH = 16384

NC = 2
NS = 16
NW = NC * NS
B_PER_W = BATCH // NW
ROWS_PER_CHUNK = 4
CHUNK = ROWS_PER_CHUNK * N_FIELDS
NCHUNKS = B_PER_W // ROWS_PER_CHUNK
NBUF = 2
NGRP = NCHUNKS // NBUF
LANES = 16


def _tec_body(x_hbm, tbl_hbm, out_hbm, x_v, idx_v, col_v, gbuf, acc,
              ld_sem, g_sems):
  wid = lax.axis_index("s") * NC + lax.axis_index("c")

  pltpu.async_copy(x_hbm.at[pl.ds(wid * NCHUNKS, NCHUNKS)], x_v,
                   ld_sem).wait()

  off0 = lax.iota(jnp.int32, LANES) * VOCAB
  off1 = off0 + 10 * VOCAB
  three = jnp.full((LANES,), 3, jnp.int32)

  def _build_and_fire(ch, b):
    for r in range(ROWS_PER_CHUNK):
      v0 = x_v[ch, pl.ds(32 * r, LANES)] + off0
      v1 = x_v[ch, pl.ds(32 * r + 10, LANES)] + off1
      idx_v[b, pl.ds(r * N_FIELDS, LANES)] = v0
      idx_v[b, pl.ds(r * N_FIELDS + 10, LANES)] = v1
    pltpu.async_copy(tbl_hbm.at[idx_v.at[b]],
                     gbuf.at[pl.ds(b * CHUNK, CHUNK)], g_sems.at[b])

  for b in range(NBUF):
    _build_and_fire(b, b)

  def _grp(g, c):
    for b in range(NBUF):
      ch = g * NBUF + b
      pltpu.make_async_copy(tbl_hbm.at[idx_v.at[b]],
                            gbuf.at[pl.ds(b * CHUNK, CHUNK)],
                            g_sems.at[b]).wait()

      for r in range(ROWS_PER_CHUNK):
        src = b * CHUNK + r * N_FIELDS
        pos = r * N_FIELDS
        p0 = [gbuf[src + i, pl.ds(0, LANES)] for i in range(4)]
        p1 = [gbuf[src + i, pl.ds(LANES, LANES)] for i in range(4)]
        for i in range(4, N_FIELDS):
          p0[i % 4] += gbuf[src + i, pl.ds(0, LANES)]
          p1[i % 4] += gbuf[src + i, pl.ds(LANES, LANES)]
        acc[ch, pl.ds(32 * r, LANES)] = (p0[0] + p0[1]) + (p0[2] + p0[3])
        acc[ch, pl.ds(32 * r + LANES, LANES)] = (
            (p1[0] + p1[1]) + (p1[2] + p1[3]))

      @pl.when(g < NGRP - 1)
      def _():
        _build_and_fire(ch + NBUF, b)

    return c

  lax.fori_loop(0, NGRP, _grp, 0, unroll=False)

  pltpu.async_copy(acc, out_hbm.at[pl.ds(wid * NCHUNKS, NCHUNKS)],
                   ld_sem).wait()


@jax.jit
def kernel(x, tables):
  tp = jnp.pad(tables, ((0, 0), (0, 0), (0, 128 - EMB)))
  tbl128 = tp.reshape(N_FIELDS * VOCAB, 128)
  xp = jnp.pad(x, ((0, 0), (0, 32 - N_FIELDS))).reshape(BATCH // 4, 128)

  mesh = plsc.VectorSubcoreMesh(core_axis_name="c", subcore_axis_name="s")
  f = pl.kernel(
      _tec_body,
      out_type=jax.ShapeDtypeStruct((BATCH // 4, 128), jnp.float32),
      mesh=mesh,
      compiler_params=pltpu.CompilerParams(use_tc_tiling_on_sc=True),
      scratch_types=[
          pltpu.VMEM((NCHUNKS, 128), jnp.int32),
          pltpu.VMEM((NBUF, CHUNK), jnp.int32),
          pltpu.VMEM((NBUF, CHUNK), jnp.int32),
          pltpu.VMEM((NBUF * CHUNK, 128), jnp.float32),
          pltpu.VMEM((NCHUNKS, 128), jnp.float32),
          pltpu.SemaphoreType.DMA,
          pltpu.SemaphoreType.DMA((NBUF,)),
      ],
  )
  return f(xp, tbl128).reshape(BATCH, EMB)


# speedup vs baseline: 1.0734x; 1.0321x over previous
import jax
import jax.numpy as jnp
from jax import lax
from jax.experimental import pallas as pl
from jax.experimental.pallas import tpu as pltpu
from jax.experimental.pallas import tpu_sc as plsc

N_FIELDS = 26
VOCAB = 100000
EMB = 32
BATCH = 16384

NC = 2
NS = 16
NW = NC * NS
B_PER_W = BATCH // NW
CHUNK = 128
SUB = B_PER_W // CHUNK
NCHUNKS = N_FIELDS * SUB
NBUF = 4
LANES = 16


def _tec_body(idx_hbm, tbl_hbm, out_hbm, idx_v, gbuf, acc, ld_sem, g_sems):
  wid = lax.axis_index("s") * NC + lax.axis_index("c")

  pltpu.async_copy(idx_hbm.at[wid], idx_v, ld_sem).wait()

  for b in range(NBUF):
    pltpu.async_copy(tbl_hbm.at[idx_v.at[b]], gbuf.at[b], g_sems.at[b])

  zero = jnp.zeros((LANES,), jnp.float32)

  def _zero(r, c):
    for u in range(8):
      acc[r * 8 + u, pl.ds(0, LANES)] = zero
      acc[r * 8 + u, pl.ds(LANES, LANES)] = zero
    return c

  lax.fori_loop(0, B_PER_W // 8, _zero, 0, unroll=False)

  def _step(ch, b):
    pltpu.make_async_copy(tbl_hbm.at[idx_v.at[ch]], gbuf.at[b],
                          g_sems.at[b]).wait()

    base = (ch % SUB) * CHUNK

    def _accum(r, c):
      row = base + r * 4
      for u in range(4):
        g0 = gbuf[b, r * 4 + u, pl.ds(0, LANES)]
        g1 = gbuf[b, r * 4 + u, pl.ds(LANES, LANES)]
        plsc.addupdate(acc.at[row + u, pl.ds(0, LANES)], g0)
        plsc.addupdate(acc.at[row + u, pl.ds(LANES, LANES)], g1)
      return c

    lax.fori_loop(0, CHUNK // 4, _accum, 0, unroll=False)

    nxt = ch + NBUF

    @pl.when(nxt < NCHUNKS)
    def _():
      pltpu.async_copy(tbl_hbm.at[idx_v.at[nxt]], gbuf.at[b], g_sems.at[b])

  def _ring(j, c):
    for b in range(NBUF):
      _step(j * NBUF + b, b)
    return c

  lax.fori_loop(0, NCHUNKS // NBUF, _ring, 0, unroll=False)

  pltpu.async_copy(acc, out_hbm.at[pl.ds(wid * B_PER_W, B_PER_W)],
                   ld_sem).wait()


@jax.jit
def kernel(x, tables):
  tbl_flat = tables.reshape(N_FIELDS * VOCAB, EMB)

  flat_idx = x.astype(jnp.int32) + (jnp.arange(N_FIELDS, dtype=jnp.int32)
                                    * VOCAB)[None, :]
  idx = flat_idx.reshape(NW, SUB, CHUNK, N_FIELDS).transpose(0, 3, 1, 2)
  idx = idx.reshape(NW, NCHUNKS, CHUNK)

  mesh = plsc.VectorSubcoreMesh(core_axis_name="c", subcore_axis_name="s")
  f = pl.kernel(
      _tec_body,
      out_type=jax.ShapeDtypeStruct((BATCH, EMB), jnp.float32),
      mesh=mesh,
      compiler_params=pltpu.CompilerParams(use_tc_tiling_on_sc=False),
      scratch_types=[
          pltpu.VMEM((NCHUNKS, CHUNK), jnp.int32),
          pltpu.VMEM((NBUF, CHUNK, EMB), jnp.float32),
          pltpu.VMEM((B_PER_W, EMB), jnp.float32),
          pltpu.SemaphoreType.DMA,
          pltpu.SemaphoreType.DMA((NBUF,)),
      ],
  )
  return f(idx, tbl_flat)
